# baseline (speedup 1.0000x reference)
import functools

import jax
import jax.numpy as jnp
from jax import lax
from jax.experimental import pallas as pl
from jax.experimental.pallas import tpu as pltpu
from jax.experimental.pallas import tpu_sc as plsc

WIN = 64


@functools.lru_cache(maxsize=None)
def _build_call(D, T, F, B):
    info = plsc.get_sparse_core_info()
    NC, NS, L = info.num_cores, info.num_subcores, info.num_lanes
    NW = NC * NS
    assert L == 16 and B % (NW * L) == 0 and F % L == 0

    QPW = B // NW
    ROWS = QPW * WIN
    CR = 512
    NCH = ROWS // CR
    NIDX = 128
    QG = QPW // L

    mesh = plsc.VectorSubcoreMesh(core_axis_name="c", subcore_axis_name="s")

    @functools.partial(
        pl.kernel,
        mesh=mesh,
        compiler_params=pltpu.CompilerParams(
            needs_layout_passes=False, use_tc_tiling_on_sc=False),
        out_type=jax.ShapeDtypeStruct((B * WIN, F), jnp.float32),
        scratch_types=dict(
            d_v=pltpu.VMEM((QPW,), jnp.int32),
            t_v=pltpu.VMEM((QPW,), jnp.int32),
            sh_v=pltpu.VMEM((L,), jnp.int32),
            idx_buf=pltpu.VMEM((ROWS,), jnp.int32),
            fence_sp=pltpu.VMEM_SHARED((ROWS,), jnp.int32),
            rows0=pltpu.VMEM((CR, F), jnp.float32),
            rows1=pltpu.VMEM((CR, F), jnp.float32),
            gsem0=pltpu.SemaphoreType.DMA,
            gsem1=pltpu.SemaphoreType.DMA,
            osem0=pltpu.SemaphoreType.DMA,
            osem1=pltpu.SemaphoreType.DMA,
        ),
    )
    def call(tbl, didx, tidx, shv, out, *, d_v, t_v, sh_v, idx_buf, fence_sp,
             rows0, rows1, gsem0, gsem1, osem0, osem1):
        wid = lax.axis_index("s") * NC + lax.axis_index("c")
        qbase = pl.multiple_of(wid * QPW, QPW)
        obase = pl.multiple_of(wid * ROWS, ROWS)

        pltpu.sync_copy(didx.at[pl.ds(qbase, QPW)], d_v)
        pltpu.sync_copy(tidx.at[pl.ds(qbase, QPW)], t_v)
        pltpu.sync_copy(shv, sh_v)

        lane = lax.iota(jnp.int32, 16)
        sh = sh_v[...]

        def qstep(i, _):
            off = pl.multiple_of(i * L, L)
            dq = d_v[pl.ds(off, L)]
            tq = t_v[pl.ds(off, L)] + sh
            rowbase = dq * T
            pos0 = i * (L * WIN) + lane * WIN

            def wstep(w, _):
                val = tq + (w - WIN)
                r = lax.rem(val, T)
                r = jnp.where(r < 0, r + T, r)
                plsc.store_scatter(idx_buf, [pos0 + w], rowbase + r)
                return 0

            lax.fori_loop(0, WIN, wstep, 0, unroll=8)
            return 0

        lax.fori_loop(0, QG, qstep, 0)

        pltpu.sync_copy(idx_buf, fence_sp)

        def issue_gather(c, rbuf, sem):
            r0 = pl.multiple_of(c * CR, CR)
            for j in range(CR // NIDX):
                pltpu.async_copy(
                    tbl.at[idx_buf.at[pl.ds(r0 + j * NIDX, NIDX)]],
                    rbuf.at[pl.ds(j * NIDX, NIDX)],
                    sem,
                )

        def wait_gather(rbuf, sem):
            pltpu.make_async_copy(tbl.at[pl.ds(0, CR)], rbuf, sem).wait()

        def wait_scatter(rbuf, sem):
            pltpu.make_async_copy(rbuf, out.at[pl.ds(obase, CR)], sem).wait()

        issue_gather(0, rows0, gsem0)
        issue_gather(1, rows1, gsem1)

        bufs = ((rows0, gsem0, osem0), (rows1, gsem1, osem1))

        def cstep(c2, _):
            for par, (rbuf, gsem, osem) in enumerate(bufs):
                c = c2 * 2 + par
                wait_gather(rbuf, gsem)
                r0 = pl.multiple_of(c * CR, CR)
                pltpu.async_copy(rbuf, out.at[pl.ds(obase + r0, CR)], osem)

                @pl.when(c2 < NCH // 2 - 1)
                def _():
                    wait_scatter(rbuf, osem)
                    issue_gather(c + 2, rbuf, gsem)

            return 0

        lax.fori_loop(0, NCH // 2, cstep, 0)
        wait_scatter(rows0, osem0)
        wait_scatter(rows1, osem1)

    return call


def kernel(data, date_idx, time_idx, window_size):
    D, T, F = data.shape
    B = date_idx.shape[0]
    tbl = data.reshape(D * T, F)
    shv = jnp.full((16,), jnp.asarray(window_size, jnp.int32) - WIN, jnp.int32)
    out = _build_call(D, T, F, B)(tbl, date_idx.astype(jnp.int32),
                                  time_idx.astype(jnp.int32), shv)
    return out.reshape(B, WIN, F)


# speedup vs baseline: 1.2009x; 1.2009x over previous
import functools

import jax
import jax.numpy as jnp
from jax import lax
from jax.experimental import pallas as pl
from jax.experimental.pallas import tpu as pltpu
from jax.experimental.pallas import tpu_sc as plsc

WIN = 64
NBUF = 8
PAD = 8


@functools.lru_cache(maxsize=None)
def _build_call(D, T, F, B):
    info = plsc.get_sparse_core_info()
    NC, NS, L = info.num_cores, info.num_subcores, info.num_lanes
    NW = NC * NS
    assert L == 16 and B % (NW * L) == 0 and T % 8 == 0

    QPW = B // NW
    NG = QPW // NBUF
    SR = WIN + PAD

    mesh = plsc.VectorSubcoreMesh(core_axis_name="c", subcore_axis_name="s")

    scratch = dict(
        d_v=pltpu.VMEM((QPW,), jnp.int32),
        t_v=pltpu.VMEM((QPW,), jnp.int32),
        sh_v=pltpu.VMEM((L,), jnp.int32),
    )
    for k in range(NBUF):
        scratch[f"slot{k}"] = pltpu.VMEM((SR, F), jnp.float32)
        scratch[f"gsem{k}"] = pltpu.SemaphoreType.DMA
        scratch[f"osem{k}"] = pltpu.SemaphoreType.DMA

    @functools.partial(
        pl.kernel,
        mesh=mesh,
        compiler_params=pltpu.CompilerParams(
            needs_layout_passes=False, use_tc_tiling_on_sc=True),
        out_type=jax.ShapeDtypeStruct((B, WIN, F), jnp.float32),
        scratch_types=scratch,
    )
    def call(data, didx, tidx, shv, out, **scr):
        slots = [scr[f"slot{k}"] for k in range(NBUF)]
        gsems = [scr[f"gsem{k}"] for k in range(NBUF)]
        osems = [scr[f"osem{k}"] for k in range(NBUF)]
        d_v, t_v, sh_v = scr["d_v"], scr["t_v"], scr["sh_v"]

        wid = lax.axis_index("s") * NC + lax.axis_index("c")
        qbase = pl.multiple_of(wid * QPW, QPW)

        pltpu.sync_copy(didx.at[pl.ds(qbase, QPW)], d_v)
        pltpu.sync_copy(tidx.at[pl.ds(qbase, QPW)], t_v)
        pltpu.sync_copy(shv, sh_v)

        lane = lax.iota(jnp.int32, 16)
        shift = jnp.sum(jnp.where(lane == 0, sh_v[...], 0))

        def extract(vec_ref, q):
            base = pl.multiple_of((q // L) * L, L)
            v = vec_ref[pl.ds(base, L)]
            return jnp.sum(jnp.where(lane == q % L, v, 0))

        def gstep(g, _):
            q0 = g * NBUF
            per_slot = []
            for k in range(NBUF):
                q = q0 + k
                d = extract(d_v, q)
                t = extract(t_v, q)
                sm = lax.rem(t + (shift - WIN), T)
                sm = jnp.where(sm < 0, sm + T, sm)
                a0 = (sm // 8) * 8
                off8 = sm - a0
                per_slot.append((q, off8))

                @pl.when(g > 0)
                def _():
                    pltpu.make_async_copy(
                        slots[k].at[pl.ds(0, WIN)],
                        out.at[0], osems[k]).wait()

                nowrap = a0 + SR <= T

                @pl.when(nowrap)
                def _():
                    pltpu.async_copy(
                        data.at[d, pl.ds(a0, SR), :], slots[k], gsems[k])

                @pl.when(jnp.logical_not(nowrap))
                def _():
                    for j in range(SR // 8):
                        bj = a0 + 8 * j
                        bj = jnp.where(bj >= T, bj - T, bj)
                        pltpu.async_copy(
                            data.at[d, pl.ds(bj, 8), :],
                            slots[k].at[pl.ds(8 * j, 8)], gsems[k])

            for k in range(NBUF):
                q, off8 = per_slot[k]
                pltpu.make_async_copy(
                    data.at[0, pl.ds(0, SR), :], slots[k], gsems[k]).wait()
                pltpu.async_copy(
                    slots[k].at[pl.ds(off8, WIN)],
                    out.at[qbase + q], osems[k])
            return 0

        lax.fori_loop(0, NG, gstep, 0)
        for k in range(NBUF):
            pltpu.make_async_copy(
                slots[k].at[pl.ds(0, WIN)], out.at[0], osems[k]).wait()

    return call


def kernel(data, date_idx, time_idx, window_size):
    D, T, F = data.shape
    B = date_idx.shape[0]
    shv = jnp.full((16,), jnp.asarray(window_size, jnp.int32) - WIN, jnp.int32)
    return _build_call(D, T, F, B)(data, date_idx.astype(jnp.int32),
                                   time_idx.astype(jnp.int32), shv)
